# baseline (speedup 1.0000x reference)
import jax
import jax.numpy as jnp
import numpy as np
from jax.experimental import pallas as pl
from jax.experimental.pallas import tpu as pltpu

N = 10000
E = 320000
D = 128
NC = 8
AVG_D_LOG = float(np.log(32.0))


def _mlp_body(hf_ref, w0_ref, b0_ref, w1_ref, b1_ref, w2_ref, b2_ref, o_ref):
    z = jnp.maximum(hf_ref[...] @ w0_ref[...] + b0_ref[...], 0.0)
    z = jnp.maximum(z @ w1_ref[...] + b1_ref[...], 0.0)
    o_ref[...] = z @ w2_ref[...] + b2_ref[...]


def _dgn_layer(hf, src, dst, snorm_n, W, b, gamma, beta):
    m = hf[src]
    ones = jnp.ones((src.shape[0],), jnp.float32)
    deg = jax.ops.segment_sum(ones, dst, num_segments=N)
    s = jax.ops.segment_sum(m, dst, num_segments=N)
    mean = s / jnp.maximum(deg, 1.0)[:, None]
    has = (deg > 0)[:, None]
    mx = jnp.where(has, jax.ops.segment_max(m, dst, num_segments=N), 0.0)
    mn = jnp.where(has, -jax.ops.segment_max(-m, dst, num_segments=N), 0.0)
    agg = jnp.concatenate([mean, mx, mn], axis=-1)
    logd = jnp.log(deg + 1.0)[:, None]
    amp = logd / AVG_D_LOG
    att = AVG_D_LOG / jnp.maximum(logd, 1e-6)
    aggf = jnp.concatenate([agg, agg * amp, agg * att], axis=-1)
    hn = aggf @ W + b
    hn = hn * snorm_n
    mu = hn.mean(axis=0)
    var = hn.var(axis=0)
    hn = (hn - mu) / jnp.sqrt(var + 1e-5) * gamma + beta
    hn = jax.nn.relu(hn)
    return hf + hn


def kernel(g, h, e, snorm_n, snorm_e, emb, W0, b0, gamma0, beta0, W1, b1, gamma1, beta1, W2, b2, gamma2, beta2, W3, b3, gamma3, beta3, Wr0, br0, Wr1, br1, Wr2, br2):
    src, dst = g[0], g[1]
    hf = emb[h]
    for (W, b, ga, be) in [(W0, b0, gamma0, beta0), (W1, b1, gamma1, beta1),
                           (W2, b2, gamma2, beta2), (W3, b3, gamma3, beta3)]:
        hf = _dgn_layer(hf, src, dst, snorm_n, W, b, ga, be)

    w2p = jnp.zeros((32, 128), jnp.float32).at[:, :NC].set(Wr2)
    b2p = jnp.zeros((1, 128), jnp.float32).at[0, :NC].set(br2)
    BN = 1000
    out = pl.pallas_call(
        _mlp_body,
        grid=(N // BN,),
        in_specs=[
            pl.BlockSpec((BN, D), lambda i: (i, 0)),
            pl.BlockSpec((D, 64), lambda i: (0, 0)),
            pl.BlockSpec((1, 64), lambda i: (0, 0)),
            pl.BlockSpec((64, 32), lambda i: (0, 0)),
            pl.BlockSpec((1, 32), lambda i: (0, 0)),
            pl.BlockSpec((32, 128), lambda i: (0, 0)),
            pl.BlockSpec((1, 128), lambda i: (0, 0)),
        ],
        out_specs=pl.BlockSpec((BN, 128), lambda i: (i, 0)),
        out_shape=jax.ShapeDtypeStruct((N, 128), jnp.float32),
    )(hf, Wr0, br0.reshape(1, 64), Wr1, br1.reshape(1, 32), w2p, b2p)
    return out[:, :NC]


# speedup vs baseline: 3.0208x; 3.0208x over previous
import functools
import jax
import jax.numpy as jnp
import numpy as np
from jax import lax
from jax.experimental import pallas as pl
from jax.experimental.pallas import tpu as pltpu
from jax.experimental.pallas import tpu_sc as plsc

N = 10000
E = 320000
D = 128
NC = 8
AVG_D_LOG = float(np.log(32.0))

NWORK = 32
SUB = 80
NPASS = 4
NCH = SUB * NPASS
NP_PAD = NWORK * NCH
EBLK = 64
OFFW = 120
F32 = jnp.float32
I32 = jnp.int32
NEG = float(-3.0e38)
POS = float(3.0e38)


def _sc_agg_body(hf_h, srcs_h, dsts_h, off_h, out_h, st, rows, idxv, dstv,
                 offv, sem):
    c = lax.axis_index("c")
    s = lax.axis_index("s")
    wid = s * 2 + c
    n0 = wid * NCH
    iota16 = lax.iota(I32, 16)
    zv = jnp.zeros((16,), F32)
    negv = jnp.full((16,), NEG, F32)
    posv = jnp.full((16,), POS, F32)

    def extract(ref, i):
        return ref[pl.ds(i, 16)][0]

    def store_accs(base_row, accs):
        base = base_row * 384
        for j in range(24):
            st[pl.ds(base + j * 16, 16)] = accs[j]

    init_accs = tuple([zv] * 8 + [negv] * 8 + [posv] * 8)

    for p in range(NPASS):
        ns = n0 + p * SUB

        @pl.when(ns < N)
        def _pass():
            b8 = (ns // 8) * 8
            pltpu.sync_copy(off_h.at[pl.ds(b8, OFFW)], offv.at[pl.ds(0, OFFW)])
            ne = jnp.minimum(ns + SUB, N)

            e0 = extract(offv, ns - b8)
            e1 = extract(offv, jnp.maximum(ne - b8, 0))

            def init_body(n, _):
                store_accs(n, init_accs)
                return 0

            lax.fori_loop(0, SUB, init_body, 0)

            bstart = (e0 // 8) * 8
            nblk = jnp.maximum((e1 - bstart + EBLK - 1) // EBLK, 0)

            def blk_body(bi, carry):
                bs = bstart + bi * EBLK
                pltpu.sync_copy(srcs_h.at[pl.ds(bs, EBLK)], idxv)
                pltpu.sync_copy(dsts_h.at[pl.ds(bs, EBLK)],
                                dstv.at[pl.ds(0, EBLK)])
                pltpu.async_copy(hf_h.at[idxv], rows, sem).wait()
                r0 = jnp.maximum(e0 - bs, 0)
                r1 = jnp.minimum(e1 - bs, EBLK)

                def edge_body(r, ec):
                    cur = ec[0]
                    accs = ec[1:]
                    d = extract(dstv, r)

                    @pl.when(jnp.logical_and(d != cur, cur >= 0))
                    def _():
                        store_accs(cur - ns, accs)

                    reset = d != cur
                    a = [jnp.where(reset, init_accs[j], accs[j])
                         for j in range(24)]
                    for j in range(8):
                        x = rows[r, pl.ds(j * 16, 16)]
                        a[j] = a[j] + x
                        a[8 + j] = jnp.maximum(a[8 + j], x)
                        a[16 + j] = jnp.minimum(a[16 + j], x)
                    return (d,) + tuple(a)

                return lax.fori_loop(r0, r1, edge_body, carry)

            carry0 = (jnp.int32(-1),) + init_accs
            cf = lax.fori_loop(0, nblk, blk_body, carry0)

            @pl.when(cf[0] >= 0)
            def _():
                store_accs(cf[0] - ns, cf[1:])

            pltpu.sync_copy(st, out_h.at[pl.ds(ns * 384, SUB * 384)])


def _sc_agg(hf, srcs_p, dsts_p, off_p):
    mesh = plsc.VectorSubcoreMesh(core_axis_name="c", subcore_axis_name="s")
    kern = functools.partial(
        pl.kernel,
        out_type=jax.ShapeDtypeStruct((NP_PAD * 384,), F32),
        mesh=mesh,
        compiler_params=pltpu.CompilerParams(use_tc_tiling_on_sc=False),
        scratch_types=[
            pltpu.VMEM((SUB * 384,), F32),
            pltpu.VMEM((EBLK, D), F32),
            pltpu.VMEM((EBLK,), I32),
            pltpu.VMEM((EBLK + 16,), I32),
            pltpu.VMEM((OFFW + 16,), I32),
            pltpu.SemaphoreType.DMA,
        ],
    )(_sc_agg_body)
    return kern(hf, srcs_p, dsts_p, off_p)


BN = 1024


def _mm(a, b):
    return lax.dot_general(a, b, (((1,), (0,)), ((), ())),
                           precision=lax.Precision.HIGHEST,
                           preferred_element_type=F32)


def _embed_body(h_ref, emb_ref, o_ref):
    hv = h_ref[...]
    oneh = (hv == lax.broadcasted_iota(I32, (BN, D), 1)).astype(F32)
    o_ref[...] = _mm(oneh, emb_ref[...])


def _embed(hp, emb):
    return pl.pallas_call(
        _embed_body,
        grid=(NP_PAD // BN,),
        in_specs=[
            pl.BlockSpec((BN, 1), lambda i: (i, 0)),
            pl.BlockSpec((D, D), lambda i: (0, 0)),
        ],
        out_specs=pl.BlockSpec((BN, D), lambda i: (i, 0)),
        out_shape=jax.ShapeDtypeStruct((NP_PAD, D), F32),
    )(hp, emb)


def _post_body(agg_ref, deg_ref, sn_ref, w_ref, b_ref, hn_ref, st_ref):
    i = pl.program_id(0)
    a = agg_ref[...]
    deg = deg_ref[...]
    has = deg > 0
    rdeg = 1.0 / jnp.maximum(deg, 1.0)
    mean = a[:, :D] * rdeg
    mx = jnp.where(has, a[:, D:2 * D], 0.0)
    mn = jnp.where(has, a[:, 2 * D:], 0.0)
    agg = jnp.concatenate([mean, mx, mn], axis=1)
    logd = jnp.log(deg + 1.0)
    amp = logd / AVG_D_LOG
    att = AVG_D_LOG / jnp.maximum(logd, 1e-6)
    aggf = jnp.concatenate([agg, agg * amp, agg * att], axis=1)
    hn = aggf @ w_ref[...] + b_ref[...]
    hn = hn * sn_ref[...]
    rid = i * BN + lax.broadcasted_iota(I32, (BN, 1), 0)
    hn = jnp.where(rid < N, hn, 0.0)
    hn_ref[...] = hn

    @pl.when(i == 0)
    def _():
        st_ref[...] = jnp.zeros_like(st_ref)

    part = jnp.concatenate(
        [jnp.sum(hn, axis=0, keepdims=True),
         jnp.sum(hn * hn, axis=0, keepdims=True)], axis=0)
    st_ref[...] = st_ref[...] + part


def _post(agg, degp, snp, W, b):
    return pl.pallas_call(
        _post_body,
        grid=(NP_PAD // BN,),
        in_specs=[
            pl.BlockSpec((BN, 384), lambda i: (i, 0)),
            pl.BlockSpec((BN, 1), lambda i: (i, 0)),
            pl.BlockSpec((BN, 1), lambda i: (i, 0)),
            pl.BlockSpec((9 * D, D), lambda i: (0, 0)),
            pl.BlockSpec((1, D), lambda i: (0, 0)),
        ],
        out_specs=[
            pl.BlockSpec((BN, D), lambda i: (i, 0)),
            pl.BlockSpec((2, D), lambda i: (0, 0)),
        ],
        out_shape=[
            jax.ShapeDtypeStruct((NP_PAD, D), F32),
            jax.ShapeDtypeStruct((2, D), F32),
        ],
    )(agg, degp, snp, W, b.reshape(1, D))


def _apply_body(hf_ref, hn_ref, sc_ref, sh_ref, o_ref):
    o_ref[...] = hf_ref[...] + jnp.maximum(
        hn_ref[...] * sc_ref[...] + sh_ref[...], 0.0)


def _apply(hf, hn, scale, shift):
    return pl.pallas_call(
        _apply_body,
        grid=(NP_PAD // BN,),
        in_specs=[
            pl.BlockSpec((BN, D), lambda i: (i, 0)),
            pl.BlockSpec((BN, D), lambda i: (i, 0)),
            pl.BlockSpec((1, D), lambda i: (0, 0)),
            pl.BlockSpec((1, D), lambda i: (0, 0)),
        ],
        out_specs=pl.BlockSpec((BN, D), lambda i: (i, 0)),
        out_shape=jax.ShapeDtypeStruct((NP_PAD, D), F32),
    )(hf, hn, scale.reshape(1, D), shift.reshape(1, D))


def _mlp_body(hf_ref, w0_ref, b0_ref, w1_ref, b1_ref, w2_ref, b2_ref, o_ref):
    z = jnp.maximum(hf_ref[...] @ w0_ref[...] + b0_ref[...], 0.0)
    z = jnp.maximum(z @ w1_ref[...] + b1_ref[...], 0.0)
    o_ref[...] = z @ w2_ref[...] + b2_ref[...]


def _readout(hf, Wr0, br0, Wr1, br1, Wr2, br2):
    w2p = jnp.zeros((32, 128), F32).at[:, :NC].set(Wr2)
    b2p = jnp.zeros((1, 128), F32).at[0, :NC].set(br2)
    out = pl.pallas_call(
        _mlp_body,
        grid=(NP_PAD // BN,),
        in_specs=[
            pl.BlockSpec((BN, D), lambda i: (i, 0)),
            pl.BlockSpec((D, 64), lambda i: (0, 0)),
            pl.BlockSpec((1, 64), lambda i: (0, 0)),
            pl.BlockSpec((64, 32), lambda i: (0, 0)),
            pl.BlockSpec((1, 32), lambda i: (0, 0)),
            pl.BlockSpec((32, 128), lambda i: (0, 0)),
            pl.BlockSpec((1, 128), lambda i: (0, 0)),
        ],
        out_specs=pl.BlockSpec((BN, 128), lambda i: (i, 0)),
        out_shape=jax.ShapeDtypeStruct((NP_PAD, 128), F32),
    )(hf, Wr0, br0.reshape(1, 64), Wr1, br1.reshape(1, 32), w2p, b2p)
    return out[:N, :NC]


def kernel(g, h, e, snorm_n, snorm_e, emb,
           W0, b0, gamma0, beta0, W1, b1, gamma1, beta1,
           W2, b2, gamma2, beta2, W3, b3, gamma3, beta3,
           Wr0, br0, Wr1, br1, Wr2, br2):
    src, dst = g[0], g[1]
    order = jnp.argsort(dst)
    srcs = jnp.take(src, order).astype(I32)
    dsts = jnp.take(dst, order).astype(I32)
    offsets = jnp.searchsorted(dsts, jnp.arange(N + 1, dtype=I32)).astype(I32)
    deg = (offsets[1:] - offsets[:-1]).astype(F32)

    srcs_p = jnp.concatenate([srcs, jnp.zeros((128,), I32)])
    dsts_p = jnp.concatenate([dsts, jnp.full((128,), N, I32)])
    off_p = jnp.concatenate(
        [offsets, jnp.full((10112 - (N + 1),), E, I32)])
    degp = jnp.zeros((NP_PAD, 1), F32).at[:N, 0].set(deg)
    snp = jnp.zeros((NP_PAD, 1), F32).at[:N].set(snorm_n)
    hp = jnp.zeros((NP_PAD, 1), I32).at[:N, 0].set(h)

    hf = _embed(hp, emb)
    for (W, b, ga, be) in [(W0, b0, gamma0, beta0), (W1, b1, gamma1, beta1),
                           (W2, b2, gamma2, beta2), (W3, b3, gamma3, beta3)]:
        agg = _sc_agg(hf, srcs_p, dsts_p, off_p).reshape(NP_PAD, 384)
        hn, st = _post(agg, degp, snp, W, b)
        mu = st[0] / N
        var = st[1] / N - mu * mu
        scale = ga / jnp.sqrt(var + 1e-5)
        shift = be - mu * scale
        hf = _apply(hf, hn, scale, shift)

    return _readout(hf, Wr0, br0, Wr1, br1, Wr2, br2)


# speedup vs baseline: 4.7596x; 1.5756x over previous
import functools
import jax
import jax.numpy as jnp
import numpy as np
from jax import lax
from jax.experimental import pallas as pl
from jax.experimental.pallas import tpu as pltpu
from jax.experimental.pallas import tpu_sc as plsc

N = 10000
E = 320000
D = 128
NC = 8
AVG_D_LOG = float(np.log(32.0))

NWORK = 32
SUB = 80
NPASS = 4
NCH = SUB * NPASS
NP_PAD = NWORK * NCH
WWIN = 512
OFFW = 120
F32 = jnp.float32
I32 = jnp.int32
NEG = float(-3.0e38)
POS = float(3.0e38)


def _sc_agg_body(hf_h, srcs_h, off_h, out_h, st, rows, idxv, offv, sem):
    c = lax.axis_index("c")
    s = lax.axis_index("s")
    wid = s * 2 + c
    n0 = wid * NCH
    zv = jnp.zeros((16,), F32)
    negv = jnp.full((16,), NEG, F32)
    posv = jnp.full((16,), POS, F32)

    def extract(ref, i):
        return ref[pl.ds(i, 16)][0]

    def store_accs(base_row, accs):
        base = base_row * 384
        for j in range(24):
            st[pl.ds(base + j * 16, 16)] = accs[j]

    init_accs = tuple([zv] * 8 + [negv] * 8 + [posv] * 8)

    def load_window(ws):
        pltpu.sync_copy(srcs_h.at[pl.ds(ws, WWIN)], idxv)
        hs = [pltpu.async_copy(hf_h.at[idxv.at[pl.ds(k * 128, 128)]],
                               rows.at[pl.ds(k * 128, 128)], sem)
              for k in range(WWIN // 128)]
        for h in hs:
            h.wait()

    for p in range(NPASS):
        ns = n0 + p * SUB

        @pl.when(ns < N)
        def _pass():
            b8 = (ns // 8) * 8
            pltpu.sync_copy(off_h.at[pl.ds(b8, OFFW)], offv.at[pl.ds(0, OFFW)])
            ne = jnp.minimum(ns + SUB, N)
            e0 = extract(offv, ns - b8)

            def node_body(i, _):
                en0 = extract(offv, ns - b8 + i)
                en1 = extract(offv, ns - b8 + i + 1)

                def edge_body(r, accs):
                    @pl.when(jnp.logical_or((r & (WWIN - 1)) == 0, r == e0))
                    def _():
                        load_window((r // WWIN) * WWIN)

                    rl = r & (WWIN - 1)
                    a = list(accs)
                    for j in range(8):
                        x = rows[rl, pl.ds(j * 16, 16)]
                        a[j] = a[j] + x
                        a[8 + j] = jnp.maximum(a[8 + j], x)
                        a[16 + j] = jnp.minimum(a[16 + j], x)
                    return tuple(a)

                accs = lax.fori_loop(en0, en1, edge_body, init_accs)
                store_accs(i, accs)
                return 0

            lax.fori_loop(0, ne - ns, node_body, 0)
            pltpu.sync_copy(st, out_h.at[pl.ds(ns * 384, SUB * 384)])


def _sc_agg(hf, srcs_p, off_p):
    mesh = plsc.VectorSubcoreMesh(core_axis_name="c", subcore_axis_name="s")
    kern = functools.partial(
        pl.kernel,
        out_type=jax.ShapeDtypeStruct((NP_PAD * 384,), F32),
        mesh=mesh,
        compiler_params=pltpu.CompilerParams(use_tc_tiling_on_sc=False),
        scratch_types=[
            pltpu.VMEM((SUB * 384,), F32),
            pltpu.VMEM((WWIN, D), F32),
            pltpu.VMEM((WWIN,), I32),
            pltpu.VMEM((OFFW + 16,), I32),
            pltpu.SemaphoreType.DMA,
        ],
    )(_sc_agg_body)
    return kern(hf, srcs_p, off_p)


BN = 1024


def _mm(a, b):
    return lax.dot_general(a, b, (((1,), (0,)), ((), ())),
                           precision=lax.Precision.HIGHEST,
                           preferred_element_type=F32)


def _embed_body(h_ref, emb_ref, o_ref):
    hv = h_ref[...]
    oneh = (hv == lax.broadcasted_iota(I32, (BN, D), 1)).astype(F32)
    o_ref[...] = _mm(oneh, emb_ref[...])


def _embed(hp, emb):
    return pl.pallas_call(
        _embed_body,
        grid=(NP_PAD // BN,),
        in_specs=[
            pl.BlockSpec((BN, 1), lambda i: (i, 0)),
            pl.BlockSpec((D, D), lambda i: (0, 0)),
        ],
        out_specs=pl.BlockSpec((BN, D), lambda i: (i, 0)),
        out_shape=jax.ShapeDtypeStruct((NP_PAD, D), F32),
    )(hp, emb)


def _post_body(agg_ref, deg_ref, sn_ref, w_ref, b_ref, hn_ref, st_ref):
    i = pl.program_id(0)
    a = agg_ref[...]
    deg = deg_ref[...]
    has = deg > 0
    rdeg = 1.0 / jnp.maximum(deg, 1.0)
    mean = a[:, :D] * rdeg
    mx = jnp.where(has, a[:, D:2 * D], 0.0)
    mn = jnp.where(has, a[:, 2 * D:], 0.0)
    agg = jnp.concatenate([mean, mx, mn], axis=1)
    logd = jnp.log(deg + 1.0)
    amp = logd / AVG_D_LOG
    att = AVG_D_LOG / jnp.maximum(logd, 1e-6)
    aggf = jnp.concatenate([agg, agg * amp, agg * att], axis=1)
    hn = aggf @ w_ref[...] + b_ref[...]
    hn = hn * sn_ref[...]
    rid = i * BN + lax.broadcasted_iota(I32, (BN, 1), 0)
    hn = jnp.where(rid < N, hn, 0.0)
    hn_ref[...] = hn

    @pl.when(i == 0)
    def _():
        st_ref[...] = jnp.zeros_like(st_ref)

    part = jnp.concatenate(
        [jnp.sum(hn, axis=0, keepdims=True),
         jnp.sum(hn * hn, axis=0, keepdims=True)], axis=0)
    st_ref[...] = st_ref[...] + part


def _post(agg, degp, snp, W, b):
    return pl.pallas_call(
        _post_body,
        grid=(NP_PAD // BN,),
        in_specs=[
            pl.BlockSpec((BN, 384), lambda i: (i, 0)),
            pl.BlockSpec((BN, 1), lambda i: (i, 0)),
            pl.BlockSpec((BN, 1), lambda i: (i, 0)),
            pl.BlockSpec((9 * D, D), lambda i: (0, 0)),
            pl.BlockSpec((1, D), lambda i: (0, 0)),
        ],
        out_specs=[
            pl.BlockSpec((BN, D), lambda i: (i, 0)),
            pl.BlockSpec((2, D), lambda i: (0, 0)),
        ],
        out_shape=[
            jax.ShapeDtypeStruct((NP_PAD, D), F32),
            jax.ShapeDtypeStruct((2, D), F32),
        ],
    )(agg, degp, snp, W, b.reshape(1, D))


def _apply_body(hf_ref, hn_ref, sc_ref, sh_ref, o_ref):
    o_ref[...] = hf_ref[...] + jnp.maximum(
        hn_ref[...] * sc_ref[...] + sh_ref[...], 0.0)


def _apply(hf, hn, scale, shift):
    return pl.pallas_call(
        _apply_body,
        grid=(NP_PAD // BN,),
        in_specs=[
            pl.BlockSpec((BN, D), lambda i: (i, 0)),
            pl.BlockSpec((BN, D), lambda i: (i, 0)),
            pl.BlockSpec((1, D), lambda i: (0, 0)),
            pl.BlockSpec((1, D), lambda i: (0, 0)),
        ],
        out_specs=pl.BlockSpec((BN, D), lambda i: (i, 0)),
        out_shape=jax.ShapeDtypeStruct((NP_PAD, D), F32),
    )(hf, hn, scale.reshape(1, D), shift.reshape(1, D))


def _mlp_body(hf_ref, w0_ref, b0_ref, w1_ref, b1_ref, w2_ref, b2_ref, o_ref):
    z = jnp.maximum(hf_ref[...] @ w0_ref[...] + b0_ref[...], 0.0)
    z = jnp.maximum(z @ w1_ref[...] + b1_ref[...], 0.0)
    o_ref[...] = z @ w2_ref[...] + b2_ref[...]


def _readout(hf, Wr0, br0, Wr1, br1, Wr2, br2):
    w2p = jnp.zeros((32, 128), F32).at[:, :NC].set(Wr2)
    b2p = jnp.zeros((1, 128), F32).at[0, :NC].set(br2)
    out = pl.pallas_call(
        _mlp_body,
        grid=(NP_PAD // BN,),
        in_specs=[
            pl.BlockSpec((BN, D), lambda i: (i, 0)),
            pl.BlockSpec((D, 64), lambda i: (0, 0)),
            pl.BlockSpec((1, 64), lambda i: (0, 0)),
            pl.BlockSpec((64, 32), lambda i: (0, 0)),
            pl.BlockSpec((1, 32), lambda i: (0, 0)),
            pl.BlockSpec((32, 128), lambda i: (0, 0)),
            pl.BlockSpec((1, 128), lambda i: (0, 0)),
        ],
        out_specs=pl.BlockSpec((BN, 128), lambda i: (i, 0)),
        out_shape=jax.ShapeDtypeStruct((NP_PAD, 128), F32),
    )(hf, Wr0, br0.reshape(1, 64), Wr1, br1.reshape(1, 32), w2p, b2p)
    return out[:N, :NC]


def kernel(g, h, e, snorm_n, snorm_e, emb,
           W0, b0, gamma0, beta0, W1, b1, gamma1, beta1,
           W2, b2, gamma2, beta2, W3, b3, gamma3, beta3,
           Wr0, br0, Wr1, br1, Wr2, br2):
    src, dst = g[0], g[1]
    order = jnp.argsort(dst)
    srcs = jnp.take(src, order).astype(I32)
    dsts = jnp.take(dst, order).astype(I32)
    offsets = jnp.searchsorted(dsts, jnp.arange(N + 1, dtype=I32)).astype(I32)
    deg = (offsets[1:] - offsets[:-1]).astype(F32)

    srcs_p = jnp.concatenate([srcs, jnp.zeros((128,), I32)])
    off_p = jnp.concatenate(
        [offsets, jnp.full((10112 - (N + 1),), E, I32)])
    degp = jnp.zeros((NP_PAD, 1), F32).at[:N, 0].set(deg)
    snp = jnp.zeros((NP_PAD, 1), F32).at[:N].set(snorm_n)
    hp = jnp.zeros((NP_PAD, 1), I32).at[:N, 0].set(h)

    hf = _embed(hp, emb)
    for (W, b, ga, be) in [(W0, b0, gamma0, beta0), (W1, b1, gamma1, beta1),
                           (W2, b2, gamma2, beta2), (W3, b3, gamma3, beta3)]:
        agg = _sc_agg(hf, srcs_p, off_p).reshape(NP_PAD, 384)
        hn, st = _post(agg, degp, snp, W, b)
        mu = st[0] / N
        var = st[1] / N - mu * mu
        scale = ga / jnp.sqrt(var + 1e-5)
        shift = be - mu * scale
        hf = _apply(hf, hn, scale, shift)

    return _readout(hf, Wr0, br0, Wr1, br1, Wr2, br2)
